# baseline (speedup 1.0000x reference)
import jax
import jax.numpy as jnp
from jax.experimental import pallas as pl
from jax.experimental.pallas import tpu as pltpu

_N, _C, _H, _W = 64, 768, 24, 24
_HW = _H * _W
_TOPK = 128
_K = 3


def _body(x_ref, x4_ref, w_ref, cat_ref, st_ref, out_ref):
    xb = x_ref[0]
    w9 = w_ref[...]

    s = jax.lax.dot_general(w9, xb, (((1,), (0,)), ((), ())),
                            preferred_element_type=jnp.float32)

    hh = jax.lax.broadcasted_iota(jnp.int32, (1, _HW), 1) // _W
    ww = jax.lax.broadcasted_iota(jnp.int32, (1, _HW), 1) % _W
    t = jnp.zeros((1, _HW), jnp.float32)
    for kh in range(_K):
        for kw in range(_K):
            k = kh * _K + kw
            dh, dw = kh - 1, kw - 1
            off = dh * _W + dw
            sk = s[k:k + 1, :]
            if off > 0:
                shifted = jnp.concatenate(
                    [sk[:, off:], jnp.zeros((1, off), jnp.float32)], axis=1)
            elif off < 0:
                shifted = jnp.concatenate(
                    [jnp.zeros((1, -off), jnp.float32), sk[:, :off]], axis=1)
            else:
                shifted = sk
            valid = ((hh + dh >= 0) & (hh + dh < _H)
                     & (ww + dw >= 0) & (ww + dw < _W))
            t = t + jnp.where(valid, shifted, 0.0)

    te = jnp.exp(t)
    tn = te / jnp.sum(te)

    vcol = jnp.transpose(jnp.broadcast_to(tn, (8, _HW)))[:, 0:1]

    ri = jax.lax.broadcasted_iota(jnp.int32, (_HW, _HW), 0)
    ci = jax.lax.broadcasted_iota(jnp.int32, (_HW, _HW), 1)
    ltb = ri < ci
    cnt = jnp.where((vcol > tn) | ((vcol == tn) & ltb), 1.0, 0.0)
    rank = jnp.sum(cnt, axis=0, keepdims=True)
    maskf = jnp.where(rank < float(_TOPK), 1.0, 0.0)

    st = (maskf - tn) + tn
    st_ref[0] = st
    outv = xb * st
    st4 = jnp.concatenate(
        [st[:, _W * r:_W * (r + 1)] for r in range(_H)], axis=0)
    out_ref[0] = x4_ref[0] * st4[None, :, :]

    ltf = jnp.where(ltb, 1.0, 0.0)
    pex = jax.lax.dot_general(maskf, ltf, (((1,), (0,)), ((), ())),
                              preferred_element_type=jnp.float32)

    mp = jnp.concatenate([jnp.broadcast_to(maskf, (4, _HW)),
                          jnp.broadcast_to(pex, (4, _HW))], axis=0)
    mpT = jnp.transpose(mp)
    mcol = mpT[:, 0:1]
    pcol = mpT[:, 4:5]

    slot = jax.lax.broadcasted_iota(
        jnp.int32, (1, _TOPK), 1).astype(jnp.float32)
    G = jnp.where((mcol > 0.5) & (pcol == slot), 1.0, 0.0)
    selT = jax.lax.dot_general(G, outv, (((0,), (1,)), ((), ())),
                               preferred_element_type=jnp.float32)
    for p in range(_TOPK):
        cat_ref[0, 0, p * _C:(p + 1) * _C] = selT[p, :]
    cat_ref[0, 0, _TOPK * _C:] = st[0, :]


_GRID_SPEC = dict(
    grid=(_N,),
    in_specs=[
        pl.BlockSpec((1, _C, _HW), lambda i: (i, 0, 0)),
        pl.BlockSpec((1, _C, _H, _W), lambda i: (i, 0, 0, 0)),
        pl.BlockSpec((16, _C), lambda i: (0, 0)),
    ],
    out_specs=[
        pl.BlockSpec((1, 1, _TOPK * _C + _HW), lambda i: (i, 0, 0)),
        pl.BlockSpec((1, 1, _HW), lambda i: (i, 0, 0)),
        pl.BlockSpec((1, _C, _H, _W), lambda i: (i, 0, 0, 0)),
    ],
)

_OUT_SHAPES = [
    jax.ShapeDtypeStruct((_N, 1, _TOPK * _C + _HW), jnp.float32),
    jax.ShapeDtypeStruct((_N, 1, _HW), jnp.float32),
    jax.ShapeDtypeStruct((_N, _C, _H, _W), jnp.float32),
]


def _run(x3, x4, w16, interpret=False):
    return pl.pallas_call(
        _body,
        out_shape=_OUT_SHAPES,
        compiler_params=pltpu.CompilerParams(
            dimension_semantics=("parallel",)),
        interpret=interpret,
        **_GRID_SPEC,
    )(x3, x4, w16)


def kernel(x, Wc):
    n, c, h, w = x.shape
    x3 = x.reshape(n, c, h * w)
    w9 = jnp.transpose(Wc[0], (1, 2, 0)).reshape(_K * _K, c)
    w16 = jnp.concatenate(
        [w9, jnp.zeros((16 - _K * _K, c), jnp.float32)], axis=0)
    cat3, st3, out = _run(x3, x, w16)
    st_mask = st3.reshape(n, 1, h, w)
    concat_out = cat3.reshape(n, _TOPK * c + h * w)
    return (concat_out, st_mask, out)


# speedup vs baseline: 2.0769x; 2.0769x over previous
import jax
import jax.numpy as jnp
from jax.experimental import pallas as pl
from jax.experimental.pallas import tpu as pltpu

_N, _C, _H, _W = 64, 768, 24, 24
_HW = _H * _W
_TOPK = 128
_K = 3


def _body(x_ref, w_ref, cat_ref, st_ref, out_ref):
    xb = x_ref[0]
    w9 = w_ref[...]

    s = jax.lax.dot_general(w9, xb, (((1,), (0,)), ((), ())),
                            preferred_element_type=jnp.float32)

    hh = jax.lax.broadcasted_iota(jnp.int32, (1, _HW), 1) // _W
    ww = jax.lax.broadcasted_iota(jnp.int32, (1, _HW), 1) % _W
    t = jnp.zeros((1, _HW), jnp.float32)
    for kh in range(_K):
        for kw in range(_K):
            k = kh * _K + kw
            dh, dw = kh - 1, kw - 1
            off = dh * _W + dw
            sk = s[k:k + 1, :]
            if off > 0:
                shifted = jnp.concatenate(
                    [sk[:, off:], jnp.zeros((1, off), jnp.float32)], axis=1)
            elif off < 0:
                shifted = jnp.concatenate(
                    [jnp.zeros((1, -off), jnp.float32), sk[:, :off]], axis=1)
            else:
                shifted = sk
            valid = ((hh + dh >= 0) & (hh + dh < _H)
                     & (ww + dw >= 0) & (ww + dw < _W))
            t = t + jnp.where(valid, shifted, 0.0)

    te = jnp.exp(t)
    tn = te / jnp.sum(te)

    vcol = jnp.transpose(jnp.broadcast_to(tn, (8, _HW)))[:, 0:1]

    ri = jax.lax.broadcasted_iota(jnp.int32, (_HW, _HW), 0)
    ci = jax.lax.broadcasted_iota(jnp.int32, (_HW, _HW), 1)
    ltb = ri < ci
    cnt = jnp.where((vcol > tn) | ((vcol == tn) & ltb), 1.0, 0.0)
    rank = jnp.sum(cnt, axis=0, keepdims=True)
    maskf = jnp.where(rank < float(_TOPK), 1.0, 0.0)

    st = (maskf - tn) + tn
    st_ref[0] = st
    outv = xb * st
    out_ref[0] = outv

    ltf = jnp.where(ltb, 1.0, 0.0)
    pex = jax.lax.dot_general(maskf, ltf, (((1,), (0,)), ((), ())),
                              preferred_element_type=jnp.float32)

    mp = jnp.concatenate([jnp.broadcast_to(maskf, (4, _HW)),
                          jnp.broadcast_to(pex, (4, _HW))], axis=0)
    mpT = jnp.transpose(mp)
    mcol = mpT[:, 0:1]
    pcol = mpT[:, 4:5]

    slot = jax.lax.broadcasted_iota(
        jnp.int32, (1, _TOPK), 1).astype(jnp.float32)
    G = jnp.where((mcol > 0.5) & (pcol == slot), 1.0, 0.0)
    selT = jax.lax.dot_general(G, outv, (((0,), (1,)), ((), ())),
                               preferred_element_type=jnp.float32)
    for p in range(_TOPK):
        cat_ref[0, 0, p * _C:(p + 1) * _C] = selT[p, :]
    cat_ref[0, 0, _TOPK * _C:] = st[0, :]


_GRID_SPEC = dict(
    grid=(_N,),
    in_specs=[
        pl.BlockSpec((1, _C, _HW), lambda i: (i, 0, 0)),
        pl.BlockSpec((16, _C), lambda i: (0, 0)),
    ],
    out_specs=[
        pl.BlockSpec((1, 1, _TOPK * _C + _HW), lambda i: (i, 0, 0)),
        pl.BlockSpec((1, 1, _HW), lambda i: (i, 0, 0)),
        pl.BlockSpec((1, _C, _HW), lambda i: (i, 0, 0)),
    ],
)


def _shapes(nb):
    return [
        jax.ShapeDtypeStruct((nb, 1, _TOPK * _C + _HW), jnp.float32),
        jax.ShapeDtypeStruct((nb, 1, _HW), jnp.float32),
        jax.ShapeDtypeStruct((nb, _C, _HW), jnp.float32),
    ]


def _run(x3, w16, interpret=False):
    nb = x3.shape[0]
    spec = dict(_GRID_SPEC)
    spec["grid"] = (nb,)
    return pl.pallas_call(
        _body,
        out_shape=_shapes(nb),
        compiler_params=pltpu.CompilerParams(
            dimension_semantics=("parallel",)),
        interpret=interpret,
        **spec,
    )(x3, w16)


_CHUNKS = 4


def kernel(x, Wc):
    n, c, h, w = x.shape
    w9 = jnp.transpose(Wc[0], (1, 2, 0)).reshape(_K * _K, c)
    w16 = jnp.concatenate(
        [w9, jnp.zeros((16 - _K * _K, c), jnp.float32)], axis=0)
    nb = n // _CHUNKS
    cats, sts, outs = [], [], []
    for ci in range(_CHUNKS):
        xc = x[ci * nb:(ci + 1) * nb]
        cat3, st3, out3 = _run(xc.reshape(nb, c, h * w), w16)
        cats.append(cat3)
        sts.append(st3)
        outs.append(out3)
    st_mask = jnp.concatenate(sts, axis=0).reshape(n, 1, h, w)
    out = jnp.concatenate(outs, axis=0).reshape(n, c, h, w)
    concat_out = jnp.concatenate(cats, axis=0).reshape(n, _TOPK * c + h * w)
    return (concat_out, st_mask, out)


# speedup vs baseline: 2.6129x; 1.2580x over previous
import functools

import jax
import jax.numpy as jnp
from jax import lax
from jax.experimental import pallas as pl
from jax.experimental.pallas import tpu as pltpu
from jax.experimental.pallas import tpu_sc as plsc

_N, _C, _H, _W = 64, 768, 24, 24
_HW = _H * _W
_TOPK = 128
_K = 3
_NV = _HW // 16


def _conv_tn(xb, w9):
    s = jax.lax.dot_general(w9, xb, (((1,), (0,)), ((), ())),
                            preferred_element_type=jnp.float32)
    hh = jax.lax.broadcasted_iota(jnp.int32, (1, _HW), 1) // _W
    ww = jax.lax.broadcasted_iota(jnp.int32, (1, _HW), 1) % _W
    t = jnp.zeros((1, _HW), jnp.float32)
    for kh in range(_K):
        for kw in range(_K):
            k = kh * _K + kw
            dh, dw = kh - 1, kw - 1
            off = dh * _W + dw
            sk = s[k:k + 1, :]
            if off > 0:
                shifted = jnp.concatenate(
                    [sk[:, off:], jnp.zeros((1, off), jnp.float32)], axis=1)
            elif off < 0:
                shifted = jnp.concatenate(
                    [jnp.zeros((1, -off), jnp.float32), sk[:, :off]], axis=1)
            else:
                shifted = sk
            valid = ((hh + dh >= 0) & (hh + dh < _H)
                     & (ww + dw >= 0) & (ww + dw < _W))
            t = t + jnp.where(valid, shifted, 0.0)
    te = jnp.exp(t)
    return te / jnp.sum(te)


def _score_body(x_ref, w_ref, tn_ref):
    tn_ref[0] = _conv_tn(x_ref[0], w_ref[...])


def _apply_body(x_ref, tn_ref, m_ref, cat_ref, st_ref, out_ref):
    xb = x_ref[0]
    tn = tn_ref[0]
    maskf = m_ref[0]

    st = (maskf - tn) + tn
    st_ref[0] = st
    outv = xb * st
    out_ref[0] = outv

    ri = jax.lax.broadcasted_iota(jnp.int32, (_HW, _HW), 0)
    ci = jax.lax.broadcasted_iota(jnp.int32, (_HW, _HW), 1)
    ltf = jnp.where(ri < ci, 1.0, 0.0)
    pex = jax.lax.dot_general(maskf, ltf, (((1,), (0,)), ((), ())),
                              preferred_element_type=jnp.float32)

    mp = jnp.concatenate([jnp.broadcast_to(maskf, (4, _HW)),
                          jnp.broadcast_to(pex, (4, _HW))], axis=0)
    mpT = jnp.transpose(mp)
    mcol = mpT[:, 0:1]
    pcol = mpT[:, 4:5]

    slot = jax.lax.broadcasted_iota(
        jnp.int32, (1, _TOPK), 1).astype(jnp.float32)
    G = jnp.where((mcol > 0.5) & (pcol == slot), 1.0, 0.0)
    selT = jax.lax.dot_general(G, outv, (((0,), (1,)), ((), ())),
                               preferred_element_type=jnp.float32)
    for p in range(_TOPK):
        cat_ref[0, 0, p * _C:(p + 1) * _C] = selT[p, :]
    cat_ref[0, 0, _TOPK * _C:] = st[0, :]


def _sc_topk_body(tn_hbm, mask_hbm, scores_v, mask_v):
    wid = lax.axis_index("s") * 2 + lax.axis_index("c")
    for r in range(2):
        row = wid * 2 + r
        pltpu.sync_copy(tn_hbm.at[row], scores_v)

        def sum_splat(x):
            return plsc.cumsum(x) + lax.rev(plsc.cumsum(lax.rev(x, (0,))),
                                            (0,)) - x

        def max_splat(x):
            return jnp.maximum(plsc.cummax(x),
                               lax.rev(plsc.cummax(lax.rev(x, (0,))), (0,)))

        def count_gt(theta):
            acc = None
            for j in range(_NV):
                v = scores_v[pl.ds(j * 16, 16)]
                c = jnp.where(v > theta, 1.0, 0.0)
                acc = c if acc is None else acc + c
            return sum_splat(acc)

        def snap_above(theta):
            acc = None
            for j in range(_NV):
                v = scores_v[pl.ds(j * 16, 16)]
                c = jnp.where(v > theta, -v, -2.0)
                acc = c if acc is None else jnp.maximum(acc, c)
            return -max_splat(acc)

        v0 = scores_v[pl.ds(0, 16)]
        zero = v0 - v0
        lo = zero
        hi = zero + 1.0
        for _ in range(32):
            mid = (lo + hi) * 0.5
            take = count_gt(mid) >= float(_TOPK)
            lo = jnp.where(take, mid, lo)
            hi = jnp.where(take, hi, mid)
        thr = snap_above(lo)
        for _ in range(5):
            adv = count_gt(thr) >= float(_TOPK)
            thr = jnp.where(adv, snap_above(thr), thr)
        need = zero + float(_TOPK) - count_gt(thr)
        run = zero
        for j in range(_NV):
            v = scores_v[pl.ds(j * 16, 16)]
            eqf = jnp.where(v == thr, 1.0, 0.0)
            pre = run + plsc.cumsum(eqf) - eqf
            tie = jnp.where(pre < need, eqf, 0.0)
            m = jnp.where(v > thr, 1.0, tie)
            mask_v[pl.ds(j * 16, 16)] = m
            run = run + sum_splat(eqf)
        pltpu.sync_copy(mask_v, mask_hbm.at[row])


def _sc_topk(tn2):
    mesh = plsc.VectorSubcoreMesh(core_axis_name="c", subcore_axis_name="s")

    @functools.partial(
        pl.kernel,
        out_type=jax.ShapeDtypeStruct((_N, _HW), jnp.float32),
        mesh=mesh,
        scratch_types=[
            pltpu.VMEM((_HW,), jnp.float32),
            pltpu.VMEM((_HW,), jnp.float32),
        ],
        compiler_params=pltpu.CompilerParams(needs_layout_passes=False),
    )
    def run(tn_hbm, mask_hbm, scores_v, mask_v):
        _sc_topk_body(tn_hbm, mask_hbm, scores_v, mask_v)

    return run(tn2)


def _run_scores(x3, w16):
    return pl.pallas_call(
        _score_body,
        out_shape=jax.ShapeDtypeStruct((_N, 1, _HW), jnp.float32),
        grid=(_N,),
        in_specs=[
            pl.BlockSpec((1, _C, _HW), lambda i: (i, 0, 0)),
            pl.BlockSpec((16, _C), lambda i: (0, 0)),
        ],
        out_specs=pl.BlockSpec((1, 1, _HW), lambda i: (i, 0, 0)),
        compiler_params=pltpu.CompilerParams(
            dimension_semantics=("parallel",)),
    )(x3, w16)


def _run_apply(x3, tn3, m3):
    return pl.pallas_call(
        _apply_body,
        out_shape=[
            jax.ShapeDtypeStruct((_N, 1, _TOPK * _C + _HW), jnp.float32),
            jax.ShapeDtypeStruct((_N, 1, _HW), jnp.float32),
            jax.ShapeDtypeStruct((_N, _C, _HW), jnp.float32),
        ],
        grid=(_N,),
        in_specs=[
            pl.BlockSpec((1, _C, _HW), lambda i: (i, 0, 0)),
            pl.BlockSpec((1, 1, _HW), lambda i: (i, 0, 0)),
            pl.BlockSpec((1, 1, _HW), lambda i: (i, 0, 0)),
        ],
        out_specs=[
            pl.BlockSpec((1, 1, _TOPK * _C + _HW), lambda i: (i, 0, 0)),
            pl.BlockSpec((1, 1, _HW), lambda i: (i, 0, 0)),
            pl.BlockSpec((1, _C, _HW), lambda i: (i, 0, 0)),
        ],
        compiler_params=pltpu.CompilerParams(
            dimension_semantics=("parallel",)),
    )(x3, tn3, m3)


def kernel(x, Wc):
    n, c, h, w = x.shape
    x3 = x.reshape(n, c, h * w)
    w9 = jnp.transpose(Wc[0], (1, 2, 0)).reshape(_K * _K, c)
    w16 = jnp.concatenate(
        [w9, jnp.zeros((16 - _K * _K, c), jnp.float32)], axis=0)
    tn3 = _run_scores(x3, w16)
    mask2 = _sc_topk(tn3.reshape(n, h * w))
    cat3, st3, out3 = _run_apply(x3, tn3, mask2.reshape(n, 1, h * w))
    st_mask = st3.reshape(n, 1, h, w)
    out = out3.reshape(n, c, h, w)
    concat_out = cat3.reshape(n, _TOPK * c + h * w)
    return (concat_out, st_mask, out)


# speedup vs baseline: 3.2620x; 1.2485x over previous
import jax
import jax.numpy as jnp
from jax.experimental import pallas as pl
from jax.experimental.pallas import tpu as pltpu

_N, _C, _H, _W = 64, 768, 24, 24
_HW = _H * _W
_TOPK = 128
_K = 3


def _body(x_ref, w_ref, cat_ref, st_ref, out_ref):
    xb = x_ref[0]
    w9 = w_ref[...]

    s = jax.lax.dot_general(w9, xb, (((1,), (0,)), ((), ())),
                            preferred_element_type=jnp.float32)

    hh = jax.lax.broadcasted_iota(jnp.int32, (1, _HW), 1) // _W
    ww = jax.lax.broadcasted_iota(jnp.int32, (1, _HW), 1) % _W
    t = jnp.zeros((1, _HW), jnp.float32)
    for kh in range(_K):
        for kw in range(_K):
            k = kh * _K + kw
            dh, dw = kh - 1, kw - 1
            off = dh * _W + dw
            sk = s[k:k + 1, :]
            if off > 0:
                shifted = jnp.concatenate(
                    [sk[:, off:], jnp.zeros((1, off), jnp.float32)], axis=1)
            elif off < 0:
                shifted = jnp.concatenate(
                    [jnp.zeros((1, -off), jnp.float32), sk[:, :off]], axis=1)
            else:
                shifted = sk
            valid = ((hh + dh >= 0) & (hh + dh < _H)
                     & (ww + dw >= 0) & (ww + dw < _W))
            t = t + jnp.where(valid, shifted, 0.0)

    te = jnp.exp(t)
    tn = te / jnp.sum(te)

    vcol = jnp.transpose(jnp.broadcast_to(tn, (8, _HW)))[:, 0:1]

    ri = jax.lax.broadcasted_iota(jnp.int32, (_HW, _HW), 0)
    ci = jax.lax.broadcasted_iota(jnp.int32, (_HW, _HW), 1)
    ltb = ri < ci
    cnt = jnp.where((vcol > tn) | ((vcol == tn) & ltb), 1.0, 0.0)
    rank = jnp.sum(cnt, axis=0, keepdims=True)
    maskf = jnp.where(rank < float(_TOPK), 1.0, 0.0)

    st = (maskf - tn) + tn
    st_ref[0] = st
    outv = xb * st
    out_ref[0] = outv

    ltf = jnp.where(ltb, 1.0, 0.0)
    pex = jax.lax.dot_general(maskf, ltf, (((1,), (0,)), ((), ())),
                              preferred_element_type=jnp.float32)

    mp = jnp.concatenate([jnp.broadcast_to(maskf, (4, _HW)),
                          jnp.broadcast_to(pex, (4, _HW))], axis=0)
    mpT = jnp.transpose(mp)
    mcol = mpT[:, 0:1]
    pcol = mpT[:, 4:5]

    slot = jax.lax.broadcasted_iota(
        jnp.int32, (1, _TOPK), 1).astype(jnp.float32)
    G = jnp.where((mcol > 0.5) & (pcol == slot), 1.0, 0.0)
    selT = jax.lax.dot_general(G, outv, (((0,), (1,)), ((), ())),
                               preferred_element_type=jnp.float32)
    for p in range(_TOPK):
        cat_ref[0, 0, p * _C:(p + 1) * _C] = selT[p, :]
    cat_ref[0, 0, _TOPK * _C:] = st[0, :]


_GRID_SPEC = dict(
    grid=(_N,),
    in_specs=[
        pl.BlockSpec((1, _C, _HW), lambda i: (i, 0, 0)),
        pl.BlockSpec((16, _C), lambda i: (0, 0)),
    ],
    out_specs=[
        pl.BlockSpec((1, 1, _TOPK * _C + _HW), lambda i: (i, 0, 0)),
        pl.BlockSpec((1, 1, _HW), lambda i: (i, 0, 0)),
        pl.BlockSpec((1, _C, _HW), lambda i: (i, 0, 0)),
    ],
)


def _shapes(nb):
    return [
        jax.ShapeDtypeStruct((nb, 1, _TOPK * _C + _HW), jnp.float32),
        jax.ShapeDtypeStruct((nb, 1, _HW), jnp.float32),
        jax.ShapeDtypeStruct((nb, _C, _HW), jnp.float32),
    ]


def _run(x3, w16, interpret=False):
    nb = x3.shape[0]
    spec = dict(_GRID_SPEC)
    spec["grid"] = (nb,)
    return pl.pallas_call(
        _body,
        out_shape=_shapes(nb),
        compiler_params=pltpu.CompilerParams(
            dimension_semantics=("parallel",)),
        interpret=interpret,
        **spec,
    )(x3, w16)


def kernel(x, Wc):
    n, c, h, w = x.shape
    x3 = x.reshape(n, c, h * w)
    w9 = jnp.transpose(Wc[0], (1, 2, 0)).reshape(_K * _K, c)
    w16 = jnp.concatenate(
        [w9, jnp.zeros((16 - _K * _K, c), jnp.float32)], axis=0)
    cat3, st3, out3 = _run(x3, w16)
    st_mask = st3.reshape(n, 1, h, w)
    out = out3.reshape(n, c, h, w)
    concat_out = cat3.reshape(n, _TOPK * c + h * w)
    return (concat_out, st_mask, out)


# speedup vs baseline: 3.2882x; 1.0080x over previous
import jax
import jax.numpy as jnp
from jax.experimental import pallas as pl
from jax.experimental.pallas import tpu as pltpu

_N, _C, _H, _W = 64, 768, 24, 24
_HW = _H * _W
_TOPK = 128
_K = 3


_SPG = 2


def _body(x_ref, w_ref, cat_ref, st_ref, out_ref):
    for b in range(_SPG):
        _sample(b, x_ref, w_ref, cat_ref, st_ref, out_ref)


def _sample(b, x_ref, w_ref, cat_ref, st_ref, out_ref):
    xb = x_ref[b]
    w9 = w_ref[...]

    s = jax.lax.dot_general(w9, xb, (((1,), (0,)), ((), ())),
                            preferred_element_type=jnp.float32)

    hh = jax.lax.broadcasted_iota(jnp.int32, (1, _HW), 1) // _W
    ww = jax.lax.broadcasted_iota(jnp.int32, (1, _HW), 1) % _W
    t = jnp.zeros((1, _HW), jnp.float32)
    for kh in range(_K):
        for kw in range(_K):
            k = kh * _K + kw
            dh, dw = kh - 1, kw - 1
            off = dh * _W + dw
            sk = s[k:k + 1, :]
            if off > 0:
                shifted = jnp.concatenate(
                    [sk[:, off:], jnp.zeros((1, off), jnp.float32)], axis=1)
            elif off < 0:
                shifted = jnp.concatenate(
                    [jnp.zeros((1, -off), jnp.float32), sk[:, :off]], axis=1)
            else:
                shifted = sk
            valid = ((hh + dh >= 0) & (hh + dh < _H)
                     & (ww + dw >= 0) & (ww + dw < _W))
            t = t + jnp.where(valid, shifted, 0.0)

    te = jnp.exp(t)
    tn = te / jnp.sum(te)

    vcol = jnp.transpose(jnp.broadcast_to(tn, (8, _HW)))[:, 0:1]

    ri = jax.lax.broadcasted_iota(jnp.int32, (_HW, _HW), 0)
    ci = jax.lax.broadcasted_iota(jnp.int32, (_HW, _HW), 1)
    ltb = ri < ci
    cnt = jnp.where((vcol > tn) | ((vcol == tn) & ltb), 1.0, 0.0)
    rank = jnp.sum(cnt, axis=0, keepdims=True)
    maskf = jnp.where(rank < float(_TOPK), 1.0, 0.0)

    st = (maskf - tn) + tn
    st_ref[b] = st
    outv = xb * st
    out_ref[b] = outv

    ltf = jnp.where(ltb, 1.0, 0.0)
    pex = jax.lax.dot_general(maskf, ltf, (((1,), (0,)), ((), ())),
                              preferred_element_type=jnp.float32)

    mp = jnp.concatenate([jnp.broadcast_to(maskf, (4, _HW)),
                          jnp.broadcast_to(pex, (4, _HW))], axis=0)
    mpT = jnp.transpose(mp)
    mcol = mpT[:, 0:1]
    pcol = mpT[:, 4:5]

    slot = jax.lax.broadcasted_iota(
        jnp.int32, (1, _TOPK), 1).astype(jnp.float32)
    G = jnp.where((mcol > 0.5) & (pcol == slot), 1.0, 0.0)
    selT = jax.lax.dot_general(G, outv, (((0,), (1,)), ((), ())),
                               preferred_element_type=jnp.float32)
    for p in range(_TOPK):
        cat_ref[b, 0, p * _C:(p + 1) * _C] = selT[p, :]
    cat_ref[b, 0, _TOPK * _C:] = st[0, :]


_GRID_SPEC = dict(
    grid=(_N,),
    in_specs=[
        pl.BlockSpec((_SPG, _C, _HW), lambda i: (i, 0, 0)),
        pl.BlockSpec((16, _C), lambda i: (0, 0)),
    ],
    out_specs=[
        pl.BlockSpec((_SPG, 1, _TOPK * _C + _HW), lambda i: (i, 0, 0)),
        pl.BlockSpec((_SPG, 1, _HW), lambda i: (i, 0, 0)),
        pl.BlockSpec((_SPG, _C, _HW), lambda i: (i, 0, 0)),
    ],
)


def _shapes(nb):
    return [
        jax.ShapeDtypeStruct((nb, 1, _TOPK * _C + _HW), jnp.float32),
        jax.ShapeDtypeStruct((nb, 1, _HW), jnp.float32),
        jax.ShapeDtypeStruct((nb, _C, _HW), jnp.float32),
    ]


def _run(x3, w16, interpret=False):
    nb = x3.shape[0]
    spec = dict(_GRID_SPEC)
    spec["grid"] = (nb // _SPG,)
    return pl.pallas_call(
        _body,
        out_shape=_shapes(nb),
        compiler_params=pltpu.CompilerParams(
            dimension_semantics=("parallel",)),
        interpret=interpret,
        **spec,
    )(x3, w16)


def kernel(x, Wc):
    n, c, h, w = x.shape
    x3 = x.reshape(n, c, h * w)
    w9 = jnp.transpose(Wc[0], (1, 2, 0)).reshape(_K * _K, c)
    w16 = jnp.concatenate(
        [w9, jnp.zeros((16 - _K * _K, c), jnp.float32)], axis=0)
    cat3, st3, out3 = _run(x3, w16)
    st_mask = st3.reshape(n, 1, h, w)
    out = out3.reshape(n, c, h, w)
    concat_out = cat3.reshape(n, _TOPK * c + h * w)
    return (concat_out, st_mask, out)
